# Initial kernel scaffold; baseline (speedup 1.0000x reference)
import jax
import jax.numpy as jnp
from jax.experimental import pallas as pl


def kernel(input_ids, bbox, token_type_ids, position_ids, font_ids, word_emb, pos_emb, x_emb, y_emb, h_emb, w_emb, tok_emb, font_emb, gamma, beta):
    raise NotImplementedError("write your pallas kernel here")


# speedup vs baseline: 1.1291x; 1.1291x over previous
import functools

import jax
import jax.numpy as jnp
from jax import lax
from jax.experimental import pallas as pl
from jax.experimental.pallas import tpu as pltpu
from jax.experimental.pallas import tpu_sc as plsc

N = 32768
H = 768
L = 16
HV = H // L
NC, NS = 2, 16
NW = NC * NS
NPW = N // NW
T = 32
NCHUNK = NPW // T
INV_H = 1.0 / H
EPS = 1e-12


def _rsqrt(x):
    xi = lax.bitcast_convert_type(x, jnp.int32)
    y = lax.bitcast_convert_type(0x5F3759DF - (xi >> 1), jnp.float32)
    for _ in range(3):
        y = y * (1.5 - 0.5 * x * y * y)
    return y


def _reduce_splat(v):
    dnums = lax.GatherDimensionNumbers(
        offset_dims=(), collapsed_slice_dims=(0,), start_index_map=(0,))
    for off in (8, 4, 2, 1):
        perm = lax.iota(jnp.int32, L) ^ off
        v = v + lax.gather(v, perm[:, None], dnums, (1,),
                           mode=lax.GatherScatterMode.PROMISE_IN_BOUNDS)
    return v


def _body(idx_hbm, word_hbm, pos_hbm, tok_hbm, font_hbm, x_hbm, y_hbm,
          hh_hbm, ww_hbm, gb_hbm, out_hbm,
          idx_v, acc, b1, b2, obuf, gb_v, s0, s1, s2):
    wid = lax.axis_index("s") * NC + lax.axis_index("c")
    base = wid * NPW
    pltpu.sync_copy(gb_hbm, gb_v)
    pltpu.sync_copy(idx_hbm.at[:, pl.ds(base, NPW)], idx_v)

    plan = [
        (word_hbm, 0, acc, s0),
        (pos_hbm, 1, b1, s1),
        (tok_hbm, 2, b2, s2),
        (font_hbm, 3, b1, s1),
        (x_hbm, 4, b2, s2),
        (y_hbm, 5, b1, s1),
        (x_hbm, 6, b2, s2),
        (y_hbm, 7, b1, s1),
        (hh_hbm, 8, b2, s2),
        (ww_hbm, 9, b1, s1),
    ]

    def accumulate(buf):
        def row(t, _):
            for i in range(HV):
                sl = pl.ds(i * L, L)
                acc[t, sl] += buf[t, sl]
            return 0
        lax.fori_loop(0, T, row, 0)

    def chunk(c, _):
        tbase = base + c * T

        def start(k):
            tab, row, buf, sem = plan[k]
            return pltpu.async_copy(
                tab.at[idx_v.at[row, pl.ds(c * T, T)]], buf, sem)

        copies = [start(0), start(1), start(2)]
        copies[0].wait()
        for k in range(1, 10):
            copies[k].wait()
            accumulate(plan[k][2])
            if k + 2 < 10:
                copies.append(start(k + 2))

        def norm(t, _):
            s = jnp.zeros((L,), jnp.float32)
            q = jnp.zeros((L,), jnp.float32)
            for i in range(HV):
                v = acc[t, pl.ds(i * L, L)]
                s = s + v
                q = q + v * v
            mu = _reduce_splat(s) * INV_H
            var = _reduce_splat(q) * INV_H - mu * mu
            rstd = _rsqrt(var + EPS)
            for i in range(HV):
                sl = pl.ds(i * L, L)
                v = acc[t, sl]
                obuf[t, sl] = (v - mu) * rstd * gb_v[0, sl] + gb_v[1, sl]
            return 0
        lax.fori_loop(0, T, norm, 0)
        pltpu.sync_copy(obuf, out_hbm.at[pl.ds(tbase, T)])
        return 0

    lax.fori_loop(0, NCHUNK, chunk, 0)


@functools.cache
def _build():
    mesh = plsc.VectorSubcoreMesh(core_axis_name="c", subcore_axis_name="s",
                                  num_cores=NC, num_subcores=NS)
    return pl.kernel(
        _body,
        out_type=jax.ShapeDtypeStruct((N, H), jnp.float32),
        mesh=mesh,
        scratch_types=[
            pltpu.VMEM((10, NPW), jnp.int32),
            pltpu.VMEM((T, H), jnp.float32),
            pltpu.VMEM((T, H), jnp.float32),
            pltpu.VMEM((T, H), jnp.float32),
            pltpu.VMEM((T, H), jnp.float32),
            pltpu.VMEM((2, H), jnp.float32),
            pltpu.SemaphoreType.DMA,
            pltpu.SemaphoreType.DMA,
            pltpu.SemaphoreType.DMA,
        ],
    )


def kernel(input_ids, bbox, token_type_ids, position_ids, font_ids,
           word_emb, pos_emb, x_emb, y_emb, h_emb, w_emb, tok_emb, font_emb,
           gamma, beta):
    B, S = input_ids.shape
    i32 = jnp.int32
    ids = input_ids.reshape(N).astype(i32)
    pos_idx = jnp.broadcast_to(position_ids, (B, S)).reshape(N).astype(i32)
    tok_idx = token_type_ids.reshape(N).astype(i32)
    font_idx = font_ids.reshape(N).astype(i32)
    bb = bbox.astype(i32)
    left = bb[:, :, 0].reshape(N)
    upper = bb[:, :, 1].reshape(N)
    right = bb[:, :, 2].reshape(N)
    lower = bb[:, :, 3].reshape(N)
    idx_all = jnp.stack([ids, pos_idx, tok_idx, font_idx, left, upper,
                         right, lower, lower - upper, right - left])
    gb = jnp.stack([gamma, beta])
    out = _build()(idx_all, word_emb, pos_emb, tok_emb, font_emb,
                   x_emb, y_emb, h_emb, w_emb, gb)
    return out.reshape(B, S, H)
